# Initial kernel scaffold; baseline (speedup 1.0000x reference)
import jax
import jax.numpy as jnp
from jax.experimental import pallas as pl


def kernel(x, edge_index, W, b):
    raise NotImplementedError("write your pallas kernel here")


# speedup vs baseline: 3.1237x; 3.1237x over previous
import functools

import jax
import jax.numpy as jnp
from jax import lax
from jax.experimental import pallas as pl
from jax.experimental.pallas import tpu as pltpu
from jax.experimental.pallas import tpu_sc as plsc

N_NODES = 10000
D_IN = 128
N_EDGES = 320000

NC = 2
NS = 16
NW = NC * NS

CHUNK = 128
EPT = 10240
NCHUNK = EPT // CHUNK
E_PAD = EPT * NW
NP = 10112
RPT = NP // NS
DW = 144

_mesh = plsc.VectorSubcoreMesh(core_axis_name="c", subcore_axis_name="s")


@functools.partial(
    pl.kernel,
    out_type=jax.ShapeDtypeStruct((NC, NP, DW), jnp.float32),
    mesh=_mesh,
    compiler_params=pltpu.CompilerParams(use_tc_tiling_on_sc=False),
    scratch_types=[
        pltpu.VMEM((NCHUNK, CHUNK), jnp.int32),
        pltpu.VMEM((NCHUNK, CHUNK), jnp.int32),
        pltpu.VMEM((CHUNK, DW), jnp.float32),
        pltpu.VMEM_SHARED((NP, DW), jnp.float32),
        pltpu.SemaphoreType.DMA,
    ],
)
def _sc_aggregate(xa_hbm, rows_hbm, cols_hbm, zeros_hbm, out_hbm,
                  rows_v, cols_v, gbuf, acc_sh, sem):
    c = lax.axis_index("c")
    s = lax.axis_index("s")
    w = s * NC + c
    lo = s * RPT

    pltpu.sync_copy(zeros_hbm, acc_sh.at[pl.ds(lo, RPT)])
    pltpu.sync_copy(rows_hbm.at[w], rows_v)
    pltpu.sync_copy(cols_hbm.at[w], cols_v)
    plsc.subcore_barrier()

    def body(j, carry):
        pltpu.async_copy(xa_hbm.at[rows_v.at[j]], gbuf, sem).wait()
        pltpu.sync_copy(gbuf, acc_sh.at[cols_v.at[j]], add=True)
        return carry

    lax.fori_loop(0, NCHUNK, body, 0)

    plsc.subcore_barrier()
    pltpu.sync_copy(acc_sh.at[pl.ds(lo, RPT)], out_hbm.at[c, pl.ds(lo, RPT)])


def _tc_dense_body(agg_ref, x_ref, w_ref, b_ref, o_ref):
    a = agg_ref[0] + agg_ref[1]
    feat = a[:, :D_IN]
    deg = jnp.maximum(a[:, D_IN:D_IN + 1], 1.0)
    h = feat / deg + x_ref[...]
    o = jnp.dot(h, w_ref[...], preferred_element_type=jnp.float32,
                precision=lax.Precision.HIGHEST)
    o_ref[...] = jnp.maximum(o + b_ref[...], 0.0)


_BLK = 1000


def kernel(x, edge_index, W, b):
    row = edge_index[0]
    col = edge_index[1]
    pad = E_PAD - N_EDGES
    rows = jnp.concatenate([row, jnp.zeros((pad,), jnp.int32)]).reshape(
        NW, NCHUNK, CHUNK)
    cols = jnp.concatenate([col, jnp.full((pad,), N_NODES, jnp.int32)]).reshape(
        NW, NCHUNK, CHUNK)
    ones = jnp.ones((N_NODES, 1), jnp.float32)
    zpad = jnp.zeros((N_NODES, DW - D_IN - 1), jnp.float32)
    xa = jnp.concatenate([x, ones, zpad], axis=1)
    zeros = jnp.zeros((RPT, DW), jnp.float32)

    agg2 = _sc_aggregate(xa, rows, cols, zeros)

    grid = (N_NODES + _BLK - 1) // _BLK
    out = pl.pallas_call(
        _tc_dense_body,
        grid=(grid,),
        in_specs=[
            pl.BlockSpec((NC, _BLK, DW), lambda i: (0, i, 0)),
            pl.BlockSpec((_BLK, D_IN), lambda i: (i, 0)),
            pl.BlockSpec((D_IN, D_IN), lambda i: (0, 0)),
            pl.BlockSpec((1, D_IN), lambda i: (0, 0)),
        ],
        out_specs=pl.BlockSpec((_BLK, D_IN), lambda i: (i, 0)),
        out_shape=jax.ShapeDtypeStruct((N_NODES, D_IN), jnp.float32),
    )(agg2, x, W, b.reshape(1, D_IN))
    return out
